# baseline (speedup 1.0000x reference)
import functools

import jax
import jax.numpy as jnp
from jax import lax
from jax.experimental import pallas as pl
from jax.experimental.pallas import tpu as pltpu
from jax.experimental.pallas import tpu_sc as plsc

HIDDEN = 256
OUT = 256
HEADS = 4
HEAD_DIM = OUT // HEADS
N = 50000
G = 128

BLK = 512
NP = 50688
NBLK = NP // BLK

NW = 32
GPW = G // NW
CH = 128


def _tc_body(h_ref, b_ref, w1_ref, b1_ref, w2_ref, b2_ref, pj_ref,
             whp_ref, invx_ref, meta_ref, se_ref):
    i = pl.program_id(0)

    @pl.when(i == 0)
    def _init():
        se_ref[...] = jnp.zeros_like(se_ref)

    hb = h_ref[...]
    hid = jnp.tanh(
        jnp.dot(hb, w1_ref[...], preferred_element_type=jnp.float32)
        + b1_ref[...])
    es = jnp.exp(
        jnp.dot(hid, w2_ref[...], preferred_element_type=jnp.float32)
        + b2_ref[...])
    hp = jnp.dot(hb, pj_ref[...], preferred_element_type=jnp.float32)

    bv = b_ref[0]
    oh = (lax.broadcasted_iota(jnp.int32, (G, BLK), 0) == bv
          ).astype(jnp.float32)

    e8 = (lax.broadcasted_iota(jnp.int32, (8, OUT), 0)
          == lax.broadcasted_iota(jnp.int32, (8, OUT), 1) // HEAD_DIM
          ).astype(jnp.float32)
    esx = jnp.dot(es, e8, preferred_element_type=jnp.float32,
                  precision=lax.Precision.HIGHEST)
    whp_ref[...] = hp * esx

    se_ref[...] += jnp.dot(oh, es, preferred_element_type=jnp.float32)

    @pl.when(i == NBLK - 1)
    def _fin():
        se = se_ref[...]
        inv = 1.0 / jnp.clip(se, 1e-10, None)
        invx_ref[...] = jnp.dot(inv, e8, preferred_element_type=jnp.float32,
                                precision=lax.Precision.HIGHEST)
        tri = (lax.broadcasted_iota(jnp.int32, (G, G), 1)
               < lax.broadcasted_iota(jnp.int32, (G, G), 0)).astype(jnp.float32)
        starts = jnp.dot(tri, se, preferred_element_type=jnp.float32,
                         precision=lax.Precision.HIGHEST)[:, 4:5]
        cnts = se[:, 4:5]
        col = lax.broadcasted_iota(jnp.int32, (G, 16), 1)
        metaf = jnp.where(col == 0, starts, jnp.where(col == 1, cnts, 0.0))
        meta_ref[...] = metaf.astype(jnp.int32)


def _sc_body(whp_hbm, invx_hbm, meta_hbm, pb_hbm, out_hbm,
             meta_v, invx_v, pb_v, rows_v, obuf_v):
    wid = lax.axis_index("s") * 2 + lax.axis_index("c")
    gbase = wid * GPW
    base8 = pl.multiple_of((wid // 2) * 8, 8)
    off = (wid % 2) * GPW
    pltpu.sync_copy(meta_hbm.at[pl.ds(base8, 8)], meta_v)
    pltpu.sync_copy(invx_hbm.at[pl.ds(base8, 8)], invx_v)
    pltpu.sync_copy(pb_hbm, pb_v)

    for gl in range(GPW):
        mrow = meta_v[off + gl]
        start = mrow[0]
        cnt = mrow[1]
        a0 = (start // 8) * 8
        lead = start - a0
        nch = (cnt + lead + CH - 1) // CH

        def chunk_body(c, acc):
            row0 = pl.multiple_of(a0 + c * CH, 8)
            pltpu.sync_copy(whp_hbm.at[pl.ds(row0, CH)], rows_v)
            jlo = jnp.maximum(start - row0, 0)
            jhi = jnp.minimum(start + cnt - row0, CH)

            def row_body(j, acc2):
                return tuple(acc2[k] + rows_v[j, pl.ds(k * 16, 16)]
                             for k in range(16))

            return lax.fori_loop(jlo, jhi, row_body, acc)

        zero = jnp.zeros((16,), jnp.float32)
        acc = lax.fori_loop(0, nch, chunk_body, tuple(zero for _ in range(16)))
        for k in range(16):
            obuf_v[pl.ds(k * 16, 16)] = (
                acc[k] * invx_v[off + gl, pl.ds(k * 16, 16)]
                + pb_v[pl.ds(k * 16, 16)])
        pltpu.sync_copy(
            obuf_v, out_hbm.at[pl.ds(pl.multiple_of((gbase + gl) * OUT, 8), OUT)])


@jax.jit
def kernel(h, batch, attn_W1, attn_b1, attn_W2, attn_b2, proj_W, proj_b):
    hpad = jnp.zeros((NP, HIDDEN), jnp.float32).at[:N].set(h)
    bpad = jnp.full((NP,), G, jnp.int32).at[:N].set(batch.astype(jnp.int32))
    b3 = bpad.reshape(NBLK, 1, BLK)

    w1s = jnp.transpose(attn_W1, (1, 0, 2)).reshape(HIDDEN, HEADS * HIDDEN)
    b1s = attn_b1.reshape(1, HEADS * HIDDEN)
    w2b = (attn_W2[..., 0][:, :, None] * jnp.eye(HEADS, dtype=jnp.float32)[:, None, :]
           ).reshape(HEADS * HIDDEN, HEADS)
    w2b = jnp.pad(w2b, ((0, 0), (0, 8 - HEADS)))
    b2s = jnp.pad(attn_b2[:, 0], (0, 8 - HEADS)).reshape(1, 8)
    pjs = jnp.transpose(proj_W, (1, 0, 2)).reshape(HIDDEN, OUT)
    pbf = proj_b.reshape(OUT)

    whp, invx, meta = pl.pallas_call(
        _tc_body,
        grid=(NBLK,),
        in_specs=[
            pl.BlockSpec((BLK, HIDDEN), lambda i: (i, 0)),
            pl.BlockSpec((1, 1, BLK), lambda i: (i, 0, 0)),
            pl.BlockSpec((HIDDEN, HEADS * HIDDEN), lambda i: (0, 0)),
            pl.BlockSpec((1, HEADS * HIDDEN), lambda i: (0, 0)),
            pl.BlockSpec((HEADS * HIDDEN, 8), lambda i: (0, 0)),
            pl.BlockSpec((1, 8), lambda i: (0, 0)),
            pl.BlockSpec((HIDDEN, OUT), lambda i: (0, 0)),
        ],
        out_specs=[
            pl.BlockSpec((BLK, OUT), lambda i: (i, 0)),
            pl.BlockSpec((G, OUT), lambda i: (0, 0)),
            pl.BlockSpec((G, 16), lambda i: (0, 0)),
        ],
        out_shape=[
            jax.ShapeDtypeStruct((NP, OUT), jnp.float32),
            jax.ShapeDtypeStruct((G, OUT), jnp.float32),
            jax.ShapeDtypeStruct((G, 16), jnp.int32),
        ],
        scratch_shapes=[
            pltpu.VMEM((G, 8), jnp.float32),
        ],
    )(hpad, b3, w1s, b1s, w2b, b2s, pjs)

    sc_pool = functools.partial(
        pl.kernel,
        out_type=jax.ShapeDtypeStruct((G * OUT,), jnp.float32),
        mesh=plsc.VectorSubcoreMesh(core_axis_name="c", subcore_axis_name="s"),
        scratch_types=[
            pltpu.VMEM((8, 16), jnp.int32),
            pltpu.VMEM((8, OUT), jnp.float32),
            pltpu.VMEM((OUT,), jnp.float32),
            pltpu.VMEM((CH, OUT), jnp.float32),
            pltpu.VMEM((OUT,), jnp.float32),
        ],
    )(_sc_body)

    return sc_pool(whp, invx, meta, pbf).reshape(G, OUT)


# speedup vs baseline: 1.6487x; 1.6487x over previous
import functools

import jax
import jax.numpy as jnp
from jax import lax
from jax.experimental import pallas as pl
from jax.experimental.pallas import tpu as pltpu
from jax.experimental.pallas import tpu_sc as plsc

HIDDEN = 256
OUT = 256
HEADS = 4
HEAD_DIM = OUT // HEADS
N = 50000
G = 128

BLK = 512
NP = 50176
NBLK = NP // BLK

NW = 32
GPW = G // NW
CH = 128


def _tc_body(h_ref, b_ref, w1_ref, b1_ref, w2_ref, b2_ref, pj_ref,
             whp_ref, invx_ref, meta_ref, se_ref):
    i = pl.program_id(0)

    @pl.when(i == 0)
    def _init():
        se_ref[...] = jnp.zeros_like(se_ref)

    hb = h_ref[...].astype(jnp.bfloat16)
    hid = jnp.tanh(
        jnp.dot(hb, w1_ref[...], preferred_element_type=jnp.float32)
        + b1_ref[...])
    es = jnp.exp(
        jnp.dot(hid, w2_ref[...], preferred_element_type=jnp.float32)
        + b2_ref[...])
    rowid = lax.broadcasted_iota(jnp.int32, (BLK, 8), 0) + i * BLK
    es = jnp.where(rowid < N, es, 0.0)
    hp = jnp.dot(hb, pj_ref[...], preferred_element_type=jnp.float32)

    bv = b_ref[0]
    oh = (lax.broadcasted_iota(jnp.int32, (G, BLK), 0) == bv
          ).astype(jnp.float32)

    esx = jnp.concatenate(
        [jnp.broadcast_to(es[:, i:i + 1], (BLK, HEAD_DIM)) for i in range(HEADS)],
        axis=1)
    whpf = hp * esx
    lo = lax.convert_element_type(
        lax.bitcast_convert_type(whpf[:, :128].astype(jnp.bfloat16), jnp.uint16),
        jnp.uint32)
    hi = lax.convert_element_type(
        lax.bitcast_convert_type(whpf[:, 128:].astype(jnp.bfloat16), jnp.uint16),
        jnp.uint32)
    whp_ref[...] = lo | (hi << 16)

    se_ref[...] += jnp.dot(oh, es, preferred_element_type=jnp.float32)

    @pl.when(i == NBLK - 1)
    def _fin():
        se = se_ref[...]
        inv = 1.0 / jnp.clip(se, 1e-10, None)
        invx_ref[...] = jnp.concatenate(
            [jnp.broadcast_to(inv[:, i:i + 1], (G, HEAD_DIM))
             for i in range(HEADS)], axis=1)
        tri = (lax.broadcasted_iota(jnp.int32, (G, G), 1)
               < lax.broadcasted_iota(jnp.int32, (G, G), 0)).astype(jnp.float32)
        starts = jnp.dot(tri, se, preferred_element_type=jnp.float32,
                         precision=lax.Precision.HIGHEST)[:, 4:5]
        cnts = se[:, 4:5]
        col = lax.broadcasted_iota(jnp.int32, (G, 16), 1)
        metaf = jnp.where(col == 0, starts, jnp.where(col == 1, cnts, 0.0))
        meta_ref[...] = metaf.astype(jnp.int32)


def _sc_body(whp_hbm, invx_hbm, meta_hbm, pb_hbm, out_hbm,
             meta_v, invx_v, pb_v, rows_v, obuf_v):
    wid = lax.axis_index("s") * 2 + lax.axis_index("c")
    gbase = wid * GPW
    base8 = pl.multiple_of((wid // 2) * 8, 8)
    off = (wid % 2) * GPW
    pltpu.sync_copy(meta_hbm.at[pl.ds(base8, 8)], meta_v)
    pltpu.sync_copy(invx_hbm.at[pl.ds(base8, 8)], invx_v)
    pltpu.sync_copy(pb_hbm, pb_v)

    for gl in range(GPW):
        mrow = meta_v[off + gl]
        start = mrow[0]
        cnt = mrow[1]
        a0 = (start // 8) * 8
        lead = start - a0
        nch = (cnt + lead + CH - 1) // CH

        def chunk_body(c, acc):
            row0 = pl.multiple_of(a0 + c * CH, 8)
            pltpu.sync_copy(whp_hbm.at[pl.ds(row0, CH)], rows_v)
            jlo = jnp.maximum(start - row0, 0)
            jhi = jnp.minimum(start + cnt - row0, CH)

            def row_body(j, acc2):
                out = list(acc2)
                for k in range(8):
                    w = rows_v[j, pl.ds(k * 16, 16)]
                    a = plsc.bitcast(w << 16, jnp.float32)
                    b = plsc.bitcast(w & jnp.uint32(0xFFFF0000), jnp.float32)
                    out[k] = out[k] + a
                    out[8 + k] = out[8 + k] + b
                return tuple(out)

            return lax.fori_loop(jlo, jhi, row_body, acc)

        zero = jnp.zeros((16,), jnp.float32)
        acc = lax.fori_loop(0, nch, chunk_body, tuple(zero for _ in range(16)))
        for k in range(16):
            obuf_v[pl.ds(k * 16, 16)] = (
                acc[k] * invx_v[off + gl, pl.ds(k * 16, 16)]
                + pb_v[pl.ds(k * 16, 16)])
        pltpu.sync_copy(
            obuf_v, out_hbm.at[pl.ds(pl.multiple_of((gbase + gl) * OUT, 8), OUT)])


@jax.jit
def kernel(h, batch, attn_W1, attn_b1, attn_W2, attn_b2, proj_W, proj_b):
    bpad = jnp.full((NP,), G, jnp.int32).at[:N].set(batch.astype(jnp.int32))
    b3 = bpad.reshape(NBLK, 1, BLK)

    w1s = jnp.transpose(attn_W1, (1, 0, 2)).reshape(HIDDEN, HEADS * HIDDEN).astype(jnp.bfloat16)
    b1s = attn_b1.reshape(1, HEADS * HIDDEN)
    w2b = (attn_W2[..., 0][:, :, None] * jnp.eye(HEADS, dtype=jnp.float32)[:, None, :]
           ).reshape(HEADS * HIDDEN, HEADS)
    w2b = jnp.pad(w2b, ((0, 0), (0, 8 - HEADS)))
    b2s = jnp.pad(attn_b2[:, 0], (0, 8 - HEADS)).reshape(1, 8)
    pjs = jnp.transpose(proj_W, (1, 0, 2)).reshape(HIDDEN, OUT).astype(jnp.bfloat16)
    pbf = proj_b.reshape(OUT)

    whp, invx, meta = pl.pallas_call(
        _tc_body,
        grid=(NBLK,),
        in_specs=[
            pl.BlockSpec((BLK, HIDDEN), lambda i: (i, 0)),
            pl.BlockSpec((1, 1, BLK), lambda i: (i, 0, 0)),
            pl.BlockSpec((HIDDEN, HEADS * HIDDEN), lambda i: (0, 0)),
            pl.BlockSpec((1, HEADS * HIDDEN), lambda i: (0, 0)),
            pl.BlockSpec((HEADS * HIDDEN, 8), lambda i: (0, 0)),
            pl.BlockSpec((1, 8), lambda i: (0, 0)),
            pl.BlockSpec((HIDDEN, OUT), lambda i: (0, 0)),
        ],
        out_specs=[
            pl.BlockSpec((BLK, OUT // 2), lambda i: (i, 0)),
            pl.BlockSpec((G, OUT), lambda i: (0, 0)),
            pl.BlockSpec((G, 16), lambda i: (0, 0)),
        ],
        out_shape=[
            jax.ShapeDtypeStruct((NP, OUT // 2), jnp.uint32),
            jax.ShapeDtypeStruct((G, OUT), jnp.float32),
            jax.ShapeDtypeStruct((G, 16), jnp.int32),
        ],
        scratch_shapes=[
            pltpu.VMEM((G, 8), jnp.float32),
        ],
    )(h, b3, w1s, b1s, w2b, b2s, pjs)

    sc_pool = functools.partial(
        pl.kernel,
        out_type=jax.ShapeDtypeStruct((G * OUT,), jnp.float32),
        mesh=plsc.VectorSubcoreMesh(core_axis_name="c", subcore_axis_name="s"),
        compiler_params=pltpu.CompilerParams(needs_layout_passes=False),
        scratch_types=[
            pltpu.VMEM((8, 16), jnp.int32),
            pltpu.VMEM((8, OUT), jnp.float32),
            pltpu.VMEM((OUT,), jnp.float32),
            pltpu.VMEM((CH, OUT // 2), jnp.uint32),
            pltpu.VMEM((OUT,), jnp.float32),
        ],
    )(_sc_body)

    return sc_pool(whp, invx, meta, pbf).reshape(G, OUT)
